# Initial kernel scaffold; baseline (speedup 1.0000x reference)
import jax
import jax.numpy as jnp
from jax.experimental import pallas as pl


def kernel(x_node, x_edge, edge_index, node_W, node_b, edge_W, edge_b, en_W1, en_b1, en_W2, en_b2, gru_Wih, gru_Whh, gru_bih, gru_bhh):
    raise NotImplementedError("write your pallas kernel here")


# speedup vs baseline: 1.9314x; 1.9314x over previous
import functools

import numpy as np
import jax
import jax.numpy as jnp
from jax import lax
from jax.experimental import pallas as pl
from jax.experimental.pallas import tpu as pltpu
from jax.experimental.pallas import tpu_sc as plsc

N = 10000
E = 160000
H = 32
HH = H * H
N_LAYERS = 3

CH = 128
NCHUNK = E // CH
NC, NS = 2, 16
NW = NC * NS
HALF = NCHUNK // NC
NSTRIPE = N // NS

BE = 1000
BN = 1000

_f32 = jnp.float32


def _dot(a, b):
    return jnp.dot(a, b, preferred_element_type=_f32)


def _embed_node_body(x_ref, w_ref, b_ref, o_ref):
    o_ref[...] = _dot(x_ref[...], w_ref[...]) + b_ref[...]


def _embed_node(x, wT, b):
    grid = N // BN
    return pl.pallas_call(
        _embed_node_body,
        grid=(grid,),
        in_specs=[
            pl.BlockSpec((BN, x.shape[1]), lambda i: (i, 0)),
            pl.BlockSpec(wT.shape, lambda i: (0, 0)),
            pl.BlockSpec(b.shape, lambda i: (0, 0)),
        ],
        out_specs=pl.BlockSpec((BN, H), lambda i: (i, 0)),
        out_shape=jax.ShapeDtypeStruct((N, H), _f32),
    )(x, wT, b)


def _edge_feat_body(x_ref, w_ref, b_ref, w1_ref, b1_ref, o_ref):
    t = _dot(x_ref[...], w_ref[...]) + b_ref[...]
    o_ref[...] = jnp.maximum(_dot(t, w1_ref[...]) + b1_ref[...], 0.0)


def _edge_feat(x, wT, b, w1T, b1):
    grid = E // BE
    return pl.pallas_call(
        _edge_feat_body,
        grid=(grid,),
        in_specs=[
            pl.BlockSpec((BE, x.shape[1]), lambda i: (i, 0)),
            pl.BlockSpec(wT.shape, lambda i: (0, 0)),
            pl.BlockSpec(b.shape, lambda i: (0, 0)),
            pl.BlockSpec(w1T.shape, lambda i: (0, 0)),
            pl.BlockSpec(b1.shape, lambda i: (0, 0)),
        ],
        out_specs=pl.BlockSpec((BE, H), lambda i: (i, 0)),
        out_shape=jax.ShapeDtypeStruct((E, H), _f32),
    )(x, wT, b, w1T, b1)


def _msg_body(hs_ref, ef_ref, r_ref, t_ref, w2_ref, b2_ref, o_ref):
    hs = hs_ref[...]
    p = _dot(hs, r_ref[...]) * _dot(ef_ref[...], t_ref[...])
    o_ref[...] = _dot(p, w2_ref[...]) + _dot(hs, b2_ref[...])


def _msg(hs, ef, R, T, w2p, b2):
    grid = E // BE
    return pl.pallas_call(
        _msg_body,
        grid=(grid,),
        in_specs=[
            pl.BlockSpec((BE, H), lambda i: (i, 0)),
            pl.BlockSpec((BE, H), lambda i: (i, 0)),
            pl.BlockSpec(R.shape, lambda i: (0, 0)),
            pl.BlockSpec(T.shape, lambda i: (0, 0)),
            pl.BlockSpec(w2p.shape, lambda i: (0, 0)),
            pl.BlockSpec(b2.shape, lambda i: (0, 0)),
        ],
        out_specs=pl.BlockSpec((BE, H), lambda i: (i, 0)),
        out_shape=jax.ShapeDtypeStruct((E, H), _f32),
    )(hs, ef, R, T, w2p, b2)


def _gru_body(p0_ref, p1_ref, d0_ref, d1_ref, h_ref,
              wir_ref, wiz_ref, win_ref, whr_ref, whz_ref, whn_ref,
              br_ref, bz_ref, bin_ref, bhn_ref, o_ref):
    deg = jnp.maximum(d0_ref[...] + d1_ref[...], 1.0)
    m = jnp.maximum((p0_ref[...] + p1_ref[...]) / deg, 0.0)
    h = h_ref[...]
    r = jax.nn.sigmoid(_dot(m, wir_ref[...]) + _dot(h, whr_ref[...]) + br_ref[...])
    z = jax.nn.sigmoid(_dot(m, wiz_ref[...]) + _dot(h, whz_ref[...]) + bz_ref[...])
    n = jnp.tanh(_dot(m, win_ref[...]) + bin_ref[...]
                 + r * (_dot(h, whn_ref[...]) + bhn_ref[...]))
    o_ref[...] = (1.0 - z) * n + z * h


def _gru(p0, p1, d0, d1, h, mats, biases):
    grid = N // BN
    blk = pl.BlockSpec((BN, H), lambda i: (i, 0))
    full = lambda a: pl.BlockSpec(a.shape, lambda i: (0, 0))
    return pl.pallas_call(
        _gru_body,
        grid=(grid,),
        in_specs=[blk] * 5 + [full(m) for m in mats] + [full(b) for b in biases],
        out_specs=blk,
        out_shape=jax.ShapeDtypeStruct((N, H), _f32),
    )(p0, p1, d0, d1, h, *mats, *biases)


@functools.cache
def _build_sc_kernels():
    mesh = plsc.VectorSubcoreMesh(core_axis_name="c", subcore_axis_name="s",
                                  num_cores=NC, num_subcores=NS)

    @functools.partial(
        pl.kernel, mesh=mesh,
        out_type=jax.ShapeDtypeStruct((E, H), _f32),
        scratch_types=[
            pltpu.VMEM((CH,), jnp.int32),
            pltpu.VMEM((CH, H), _f32),
            pltpu.SemaphoreType.DMA,
        ],
        compiler_params=pltpu.CompilerParams(use_tc_tiling_on_sc=False),
    )
    def sc_gather(h_hbm, src_hbm, out_hbm, idx_v, rows_v, sem):
        wid = lax.axis_index("s") * NC + lax.axis_index("c")
        lo = (wid * NCHUNK) // NW
        hi = ((wid + 1) * NCHUNK) // NW

        def body(j, carry):
            pltpu.sync_copy(src_hbm.at[j], idx_v)
            pltpu.async_copy(h_hbm.at[idx_v], rows_v, sem).wait()
            pltpu.sync_copy(rows_v, out_hbm.at[pl.ds(j * CH, CH)])
            return carry

        lax.fori_loop(lo, hi, body, 0)

    @functools.partial(
        pl.kernel, mesh=mesh,
        out_type=jax.ShapeDtypeStruct((NC * N, H), _f32),
        scratch_types=[
            pltpu.VMEM((CH,), jnp.int32),
            pltpu.VMEM((CH, H), _f32),
            pltpu.VMEM((NSTRIPE, H), _f32),
            pltpu.VMEM_SHARED((N, H), _f32),
            pltpu.SemaphoreType.DMA,
        ],
        compiler_params=pltpu.CompilerParams(use_tc_tiling_on_sc=False),
    )
    def sc_scatter(vals_hbm, dst_hbm, zeros_hbm, out_hbm, idx_v, val_v, buf_v,
                   acc, sem):
        c = lax.axis_index("c")
        s = lax.axis_index("s")
        pltpu.sync_copy(zeros_hbm.at[pl.ds(s * NSTRIPE, NSTRIPE)], buf_v)
        pltpu.sync_copy(buf_v, acc.at[pl.ds(s * NSTRIPE, NSTRIPE)])
        plsc.subcore_barrier()

        lo = c * HALF + (s * HALF) // NS
        hi = c * HALF + ((s + 1) * HALF) // NS

        def body(j, carry):
            pltpu.sync_copy(dst_hbm.at[j], idx_v)
            pltpu.sync_copy(vals_hbm.at[pl.ds(j * CH, CH)], val_v)
            pltpu.sync_copy(val_v, acc.at[idx_v], add=True)
            return carry

        lax.fori_loop(lo, hi, body, 0)
        plsc.subcore_barrier()
        pltpu.sync_copy(acc.at[pl.ds(s * NSTRIPE, NSTRIPE)], buf_v)
        pltpu.sync_copy(buf_v, out_hbm.at[pl.ds(c * N + s * NSTRIPE, NSTRIPE)])

    return sc_gather, sc_scatter


def _sc_gather(h, src2d):
    return _build_sc_kernels()[0](h, src2d)


def _sc_scatter(vals, dst2d, zeros_nh):
    return _build_sc_kernels()[1](vals, dst2d, zeros_nh)


_R_EXP = np.kron(np.eye(H), np.ones((1, H))).astype(np.float32)
_T_EXP = np.kron(np.ones((1, H)), np.eye(H)).astype(np.float32)


def kernel(x_node, x_edge, edge_index, node_W, node_b, edge_W, edge_b,
           en_W1, en_b1, en_W2, en_b2, gru_Wih, gru_Whh, gru_bih, gru_bhh):
    src = edge_index[0].reshape(NCHUNK, CH)
    dst = edge_index[1].reshape(NCHUNK, CH)

    node_WT = node_W.T
    node_b2d = node_b.reshape(1, H)
    edge_WT = edge_W.T
    edge_b2d = edge_b.reshape(1, H)
    en_W1T = en_W1.T
    en_b12d = en_b1.reshape(1, H)
    w2p = en_W2.reshape(H, H, H).transpose(0, 2, 1).reshape(HH, H)
    b2 = en_b2.reshape(H, H)
    R = jnp.asarray(_R_EXP)
    T = jnp.asarray(_T_EXP)

    wir = gru_Wih[0:H].T
    wiz = gru_Wih[H:2 * H].T
    win = gru_Wih[2 * H:3 * H].T
    whr = gru_Whh[0:H].T
    whz = gru_Whh[H:2 * H].T
    whn = gru_Whh[2 * H:3 * H].T
    br = (gru_bih[0:H] + gru_bhh[0:H]).reshape(1, H)
    bz = (gru_bih[H:2 * H] + gru_bhh[H:2 * H]).reshape(1, H)
    bin_ = gru_bih[2 * H:3 * H].reshape(1, H)
    bhn = gru_bhh[2 * H:3 * H].reshape(1, H)
    mats = (wir, wiz, win, whr, whz, whn)
    biases = (br, bz, bin_, bhn)

    zeros_nh = jnp.zeros((N, H), _f32)
    ones_eh = jnp.ones((E, H), _f32)

    h = _embed_node(x_node, node_WT, node_b2d)
    ef = _edge_feat(x_edge, edge_WT, edge_b2d, en_W1T, en_b12d)

    degp = _sc_scatter(ones_eh, dst, zeros_nh)
    d0, d1 = degp[:N], degp[N:]

    for _ in range(N_LAYERS):
        hs = _sc_gather(h, src)
        msg = _msg(hs, ef, R, T, w2p, b2)
        mp = _sc_scatter(msg, dst, zeros_nh)
        h = _gru(mp[:N], mp[N:], d0, d1, h, mats, biases)
    return h


# speedup vs baseline: 2.5469x; 1.3187x over previous
import functools

import numpy as np
import jax
import jax.numpy as jnp
from jax import lax
from jax.experimental import pallas as pl
from jax.experimental.pallas import tpu as pltpu
from jax.experimental.pallas import tpu_sc as plsc

N = 10000
E = 160000
H = 32
HH = H * H
N_LAYERS = 3

CH = 128
NCHUNK = E // CH
NC, NS = 2, 16
NW = NC * NS
HALF = NCHUNK // NC
NSTRIPE = N // NS

BE = 1000
BN = 1000

_f32 = jnp.float32


def _dot(a, b):
    return jnp.dot(a, b, preferred_element_type=_f32)


def _embed_node_body(x_ref, w_ref, b_ref, o_ref):
    o_ref[...] = _dot(x_ref[...], w_ref[...]) + b_ref[...]


def _embed_node(x, wT, b):
    grid = N // BN
    return pl.pallas_call(
        _embed_node_body,
        grid=(grid,),
        in_specs=[
            pl.BlockSpec((BN, x.shape[1]), lambda i: (i, 0)),
            pl.BlockSpec(wT.shape, lambda i: (0, 0)),
            pl.BlockSpec(b.shape, lambda i: (0, 0)),
        ],
        out_specs=pl.BlockSpec((BN, H), lambda i: (i, 0)),
        out_shape=jax.ShapeDtypeStruct((N, H), _f32),
    )(x, wT, b)


def _edge_feat_body(x_ref, w_ref, b_ref, w1_ref, b1_ref, o_ref):
    t = _dot(x_ref[...], w_ref[...]) + b_ref[...]
    o_ref[...] = jnp.maximum(_dot(t, w1_ref[...]) + b1_ref[...], 0.0)


def _edge_feat(x, wT, b, w1T, b1):
    grid = E // BE
    return pl.pallas_call(
        _edge_feat_body,
        grid=(grid,),
        in_specs=[
            pl.BlockSpec((BE, x.shape[1]), lambda i: (i, 0)),
            pl.BlockSpec(wT.shape, lambda i: (0, 0)),
            pl.BlockSpec(b.shape, lambda i: (0, 0)),
            pl.BlockSpec(w1T.shape, lambda i: (0, 0)),
            pl.BlockSpec(b1.shape, lambda i: (0, 0)),
        ],
        out_specs=pl.BlockSpec((BE, H), lambda i: (i, 0)),
        out_shape=jax.ShapeDtypeStruct((E, H), _f32),
    )(x, wT, b, w1T, b1)


def _msg_body(hs_ref, ef_ref, w2_ref, b2c_ref, o_ref):
    gT = lax.dot_general(w2_ref[...], ef_ref[...], (((1,), (1,)), ((), ())),
                         preferred_element_type=_f32) + b2c_ref[...]
    hsT = hs_ref[...].T
    acc = gT[0:H, :] * hsT[0:1, :]
    for i in range(1, H):
        acc += gT[i * H:(i + 1) * H, :] * hsT[i:i + 1, :]
    o_ref[...] = acc.T


def _msg(hs, ef, w2, b2c):
    grid = E // BE
    return pl.pallas_call(
        _msg_body,
        grid=(grid,),
        in_specs=[
            pl.BlockSpec((BE, H), lambda i: (i, 0)),
            pl.BlockSpec((BE, H), lambda i: (i, 0)),
            pl.BlockSpec(w2.shape, lambda i: (0, 0)),
            pl.BlockSpec(b2c.shape, lambda i: (0, 0)),
        ],
        out_specs=pl.BlockSpec((BE, H), lambda i: (i, 0)),
        out_shape=jax.ShapeDtypeStruct((E, H), _f32),
    )(hs, ef, w2, b2c)


def _gru_body(p0_ref, p1_ref, d0_ref, d1_ref, h_ref,
              wir_ref, wiz_ref, win_ref, whr_ref, whz_ref, whn_ref,
              br_ref, bz_ref, bin_ref, bhn_ref, o_ref):
    deg = jnp.maximum(d0_ref[...] + d1_ref[...], 1.0)
    m = jnp.maximum((p0_ref[...] + p1_ref[...]) / deg, 0.0)
    h = h_ref[...]
    r = jax.nn.sigmoid(_dot(m, wir_ref[...]) + _dot(h, whr_ref[...]) + br_ref[...])
    z = jax.nn.sigmoid(_dot(m, wiz_ref[...]) + _dot(h, whz_ref[...]) + bz_ref[...])
    n = jnp.tanh(_dot(m, win_ref[...]) + bin_ref[...]
                 + r * (_dot(h, whn_ref[...]) + bhn_ref[...]))
    o_ref[...] = (1.0 - z) * n + z * h


def _gru(p0, p1, d0, d1, h, mats, biases):
    grid = N // BN
    blk = pl.BlockSpec((BN, H), lambda i: (i, 0))
    full = lambda a: pl.BlockSpec(a.shape, lambda i: (0, 0))
    return pl.pallas_call(
        _gru_body,
        grid=(grid,),
        in_specs=[blk] * 5 + [full(m) for m in mats] + [full(b) for b in biases],
        out_specs=blk,
        out_shape=jax.ShapeDtypeStruct((N, H), _f32),
    )(p0, p1, d0, d1, h, *mats, *biases)


@functools.cache
def _build_sc_kernels():
    mesh = plsc.VectorSubcoreMesh(core_axis_name="c", subcore_axis_name="s",
                                  num_cores=NC, num_subcores=NS)

    @functools.partial(
        pl.kernel, mesh=mesh,
        out_type=jax.ShapeDtypeStruct((E, H), _f32),
        scratch_types=[
            pltpu.VMEM((CH,), jnp.int32),
            pltpu.VMEM((CH, H), _f32),
            pltpu.SemaphoreType.DMA,
        ],
        compiler_params=pltpu.CompilerParams(use_tc_tiling_on_sc=False),
    )
    def sc_gather(h_hbm, src_hbm, out_hbm, idx_v, rows_v, sem):
        wid = lax.axis_index("s") * NC + lax.axis_index("c")
        lo = (wid * NCHUNK) // NW
        hi = ((wid + 1) * NCHUNK) // NW

        def body(j, carry):
            pltpu.sync_copy(src_hbm.at[j], idx_v)
            pltpu.async_copy(h_hbm.at[idx_v], rows_v, sem).wait()
            pltpu.sync_copy(rows_v, out_hbm.at[pl.ds(j * CH, CH)])
            return carry

        lax.fori_loop(lo, hi, body, 0)

    @functools.partial(
        pl.kernel, mesh=mesh,
        out_type=jax.ShapeDtypeStruct((NC * N, H), _f32),
        scratch_types=[
            pltpu.VMEM((CH,), jnp.int32),
            pltpu.VMEM((CH, H), _f32),
            pltpu.VMEM((NSTRIPE, H), _f32),
            pltpu.VMEM_SHARED((N, H), _f32),
            pltpu.SemaphoreType.DMA,
        ],
        compiler_params=pltpu.CompilerParams(use_tc_tiling_on_sc=False),
    )
    def sc_scatter(vals_hbm, dst_hbm, zeros_hbm, out_hbm, idx_v, val_v, buf_v,
                   acc, sem):
        c = lax.axis_index("c")
        s = lax.axis_index("s")
        pltpu.sync_copy(zeros_hbm.at[pl.ds(s * NSTRIPE, NSTRIPE)], buf_v)
        pltpu.sync_copy(buf_v, acc.at[pl.ds(s * NSTRIPE, NSTRIPE)])
        plsc.subcore_barrier()

        lo = c * HALF + (s * HALF) // NS
        hi = c * HALF + ((s + 1) * HALF) // NS

        def body(j, carry):
            pltpu.sync_copy(dst_hbm.at[j], idx_v)
            pltpu.sync_copy(vals_hbm.at[pl.ds(j * CH, CH)], val_v)
            pltpu.sync_copy(val_v, acc.at[idx_v], add=True)
            return carry

        lax.fori_loop(lo, hi, body, 0)
        plsc.subcore_barrier()
        pltpu.sync_copy(acc.at[pl.ds(s * NSTRIPE, NSTRIPE)], buf_v)
        pltpu.sync_copy(buf_v, out_hbm.at[pl.ds(c * N + s * NSTRIPE, NSTRIPE)])

    return sc_gather, sc_scatter


def _sc_gather(h, src2d):
    return _build_sc_kernels()[0](h, src2d)


def _sc_scatter(vals, dst2d, zeros_nh):
    return _build_sc_kernels()[1](vals, dst2d, zeros_nh)


def kernel(x_node, x_edge, edge_index, node_W, node_b, edge_W, edge_b,
           en_W1, en_b1, en_W2, en_b2, gru_Wih, gru_Whh, gru_bih, gru_bhh):
    src = edge_index[0].reshape(NCHUNK, CH)
    dst = edge_index[1].reshape(NCHUNK, CH)

    node_WT = node_W.T
    node_b2d = node_b.reshape(1, H)
    edge_WT = edge_W.T
    edge_b2d = edge_b.reshape(1, H)
    en_W1T = en_W1.T
    en_b12d = en_b1.reshape(1, H)
    b2c = en_b2.reshape(HH, 1)

    wir = gru_Wih[0:H].T
    wiz = gru_Wih[H:2 * H].T
    win = gru_Wih[2 * H:3 * H].T
    whr = gru_Whh[0:H].T
    whz = gru_Whh[H:2 * H].T
    whn = gru_Whh[2 * H:3 * H].T
    br = (gru_bih[0:H] + gru_bhh[0:H]).reshape(1, H)
    bz = (gru_bih[H:2 * H] + gru_bhh[H:2 * H]).reshape(1, H)
    bin_ = gru_bih[2 * H:3 * H].reshape(1, H)
    bhn = gru_bhh[2 * H:3 * H].reshape(1, H)
    mats = (wir, wiz, win, whr, whz, whn)
    biases = (br, bz, bin_, bhn)

    zeros_nh = jnp.zeros((N, H), _f32)
    ones_eh = jnp.ones((E, H), _f32)

    h = _embed_node(x_node, node_WT, node_b2d)
    ef = _edge_feat(x_edge, edge_WT, edge_b2d, en_W1T, en_b12d)

    degp = _sc_scatter(ones_eh, dst, zeros_nh)
    d0, d1 = degp[:N], degp[N:]

    for _ in range(N_LAYERS):
        hs = _sc_gather(h, src)
        msg = _msg(hs, ef, en_W2, b2c)
        mp = _sc_scatter(msg, dst, zeros_nh)
        h = _gru(mp[:N], mp[N:], d0, d1, h, mats, biases)
    return h


# speedup vs baseline: 2.7133x; 1.0653x over previous
import functools

import numpy as np
import jax
import jax.numpy as jnp
from jax import lax
from jax.experimental import pallas as pl
from jax.experimental.pallas import tpu as pltpu
from jax.experimental.pallas import tpu_sc as plsc

N = 10000
E = 160000
H = 32
HH = H * H
N_LAYERS = 3

NC, NS = 2, 16
NW = NC * NS
CH = 128
CPW = 40
NB = 4
NGROUPS = CPW // NB
NCHUNK_P = NW * CPW
E_PAD = NCHUNK_P * CH
N_ACC = 10016
NSTRIPE = N_ACC // NS

BE = 1280
BN = 1000

_f32 = jnp.float32


def _dot(a, b):
    return jnp.dot(a, b, preferred_element_type=_f32)


def _embed_node_body(x_ref, w_ref, b_ref, o_ref):
    o_ref[...] = _dot(x_ref[...], w_ref[...]) + b_ref[...]


def _embed_node(x, wT, b):
    grid = N // BN
    return pl.pallas_call(
        _embed_node_body,
        grid=(grid,),
        in_specs=[
            pl.BlockSpec((BN, x.shape[1]), lambda i: (i, 0)),
            pl.BlockSpec(wT.shape, lambda i: (0, 0)),
            pl.BlockSpec(b.shape, lambda i: (0, 0)),
        ],
        out_specs=pl.BlockSpec((BN, H), lambda i: (i, 0)),
        out_shape=jax.ShapeDtypeStruct((N, H), _f32),
    )(x, wT, b)


def _edge_feat_body(x_ref, w_ref, b_ref, w1_ref, b1_ref, o_ref):
    t = _dot(x_ref[...], w_ref[...]) + b_ref[...]
    o_ref[...] = jnp.maximum(_dot(t, w1_ref[...]) + b1_ref[...], 0.0)


def _edge_feat(x, wT, b, w1T, b1):
    grid = E_PAD // BE
    return pl.pallas_call(
        _edge_feat_body,
        grid=(grid,),
        in_specs=[
            pl.BlockSpec((BE, x.shape[1]), lambda i: (i, 0)),
            pl.BlockSpec(wT.shape, lambda i: (0, 0)),
            pl.BlockSpec(b.shape, lambda i: (0, 0)),
            pl.BlockSpec(w1T.shape, lambda i: (0, 0)),
            pl.BlockSpec(b1.shape, lambda i: (0, 0)),
        ],
        out_specs=pl.BlockSpec((BE, H), lambda i: (i, 0)),
        out_shape=jax.ShapeDtypeStruct((E_PAD, H), _f32),
    )(x, wT, b, w1T, b1)


def _msg_body(hs_ref, ef_ref, w2_ref, b2c_ref, o_ref):
    gT = lax.dot_general(w2_ref[...], ef_ref[...], (((1,), (1,)), ((), ())),
                         preferred_element_type=_f32) + b2c_ref[...]
    hsT = hs_ref[...].T
    acc = gT[0:H, :] * hsT[0:1, :]
    for i in range(1, H):
        acc += gT[i * H:(i + 1) * H, :] * hsT[i:i + 1, :]
    o_ref[...] = acc.T


def _msg(hs, ef, w2, b2c):
    grid = E_PAD // BE
    return pl.pallas_call(
        _msg_body,
        grid=(grid,),
        in_specs=[
            pl.BlockSpec((BE, H), lambda i: (i, 0)),
            pl.BlockSpec((BE, H), lambda i: (i, 0)),
            pl.BlockSpec(w2.shape, lambda i: (0, 0)),
            pl.BlockSpec(b2c.shape, lambda i: (0, 0)),
        ],
        out_specs=pl.BlockSpec((BE, H), lambda i: (i, 0)),
        out_shape=jax.ShapeDtypeStruct((E_PAD, H), _f32),
    )(hs, ef, w2, b2c)


def _gru_body(p0_ref, p1_ref, d0_ref, d1_ref, h_ref,
              wir_ref, wiz_ref, win_ref, whr_ref, whz_ref, whn_ref,
              br_ref, bz_ref, bin_ref, bhn_ref, o_ref):
    deg = jnp.maximum(d0_ref[...] + d1_ref[...], 1.0)
    m = jnp.maximum((p0_ref[...] + p1_ref[...]) / deg, 0.0)
    h = h_ref[...]
    r = jax.nn.sigmoid(_dot(m, wir_ref[...]) + _dot(h, whr_ref[...]) + br_ref[...])
    z = jax.nn.sigmoid(_dot(m, wiz_ref[...]) + _dot(h, whz_ref[...]) + bz_ref[...])
    n = jnp.tanh(_dot(m, win_ref[...]) + bin_ref[...]
                 + r * (_dot(h, whn_ref[...]) + bhn_ref[...]))
    o_ref[...] = (1.0 - z) * n + z * h


def _gru(p0, p1, d0, d1, h, mats, biases):
    grid = N // BN
    blk = pl.BlockSpec((BN, H), lambda i: (i, 0))
    full = lambda a: pl.BlockSpec(a.shape, lambda i: (0, 0))
    return pl.pallas_call(
        _gru_body,
        grid=(grid,),
        in_specs=[blk] * 5 + [full(m) for m in mats] + [full(b) for b in biases],
        out_specs=blk,
        out_shape=jax.ShapeDtypeStruct((N, H), _f32),
    )(p0, p1, d0, d1, h, *mats, *biases)


@functools.cache
def _build_sc_kernels():
    mesh = plsc.VectorSubcoreMesh(core_axis_name="c", subcore_axis_name="s",
                                  num_cores=NC, num_subcores=NS)

    @functools.partial(
        pl.kernel, mesh=mesh,
        out_type=jax.ShapeDtypeStruct((E_PAD, H), _f32),
        scratch_types=[
            pltpu.VMEM((CPW, CH), jnp.int32),
            pltpu.VMEM((NB, CH, H), _f32),
            pltpu.SemaphoreType.DMA,
            pltpu.SemaphoreType.DMA,
            pltpu.SemaphoreType.DMA,
            pltpu.SemaphoreType.DMA,
            pltpu.SemaphoreType.DMA,
            pltpu.SemaphoreType.DMA,
            pltpu.SemaphoreType.DMA,
            pltpu.SemaphoreType.DMA,
        ],
        compiler_params=pltpu.CompilerParams(use_tc_tiling_on_sc=False),
    )
    def sc_gather(h_hbm, src_hbm, out_hbm, idxall, rows, *sems):
        sg, sw = sems[:NB], sems[NB:]
        wid = lax.axis_index("s") * NC + lax.axis_index("c")
        base = wid * CPW

        pltpu.sync_copy(src_hbm.at[pl.ds(base, CPW)], idxall)

        def start_g(j, b):
            pltpu.async_copy(h_hbm.at[idxall.at[j]], rows.at[b], sg[b])

        def wait_g(b):
            pltpu.make_async_copy(h_hbm.at[idxall.at[0]], rows.at[b], sg[b]).wait()

        def start_w(j, b):
            pltpu.async_copy(rows.at[b], out_hbm.at[pl.ds((base + j) * CH, CH)],
                             sw[b])

        def wait_w(b):
            pltpu.make_async_copy(rows.at[b], out_hbm.at[pl.ds(0, CH)],
                                  sw[b]).wait()

        for b in range(NB):
            start_g(b, b)

        def body(g, carry):
            for b in range(NB):
                j = g * NB + b
                wait_g(b)
                start_w(j, b)
                wait_w(b)
                start_g(j + NB, b)
            return carry

        lax.fori_loop(0, NGROUPS - 1, body, 0)
        for b in range(NB):
            j = (NGROUPS - 1) * NB + b
            wait_g(b)
            start_w(j, b)
        for b in range(NB):
            wait_w(b)

    @functools.partial(
        pl.kernel, mesh=mesh,
        out_type=jax.ShapeDtypeStruct((NC * N_ACC, H), _f32),
        scratch_types=[
            pltpu.VMEM((CPW, CH), jnp.int32),
            pltpu.VMEM((NB, CH, H), _f32),
            pltpu.VMEM((NSTRIPE, H), _f32),
            pltpu.VMEM_SHARED((N_ACC, H), _f32),
            pltpu.SemaphoreType.DMA,
            pltpu.SemaphoreType.DMA,
            pltpu.SemaphoreType.DMA,
            pltpu.SemaphoreType.DMA,
            pltpu.SemaphoreType.DMA,
            pltpu.SemaphoreType.DMA,
            pltpu.SemaphoreType.DMA,
            pltpu.SemaphoreType.DMA,
        ],
        compiler_params=pltpu.CompilerParams(use_tc_tiling_on_sc=False),
    )
    def sc_scatter(vals_hbm, dst_hbm, zeros_hbm, out_hbm, idxall, vbuf, buf_v,
                   acc, *sems):
        sv, ss = sems[:NB], sems[NB:]
        c = lax.axis_index("c")
        s = lax.axis_index("s")
        wid = s * NC + c
        base = wid * CPW

        pltpu.sync_copy(dst_hbm.at[pl.ds(base, CPW)], idxall)
        pltpu.sync_copy(zeros_hbm.at[pl.ds(s * NSTRIPE, NSTRIPE)], buf_v)
        pltpu.sync_copy(buf_v, acc.at[pl.ds(s * NSTRIPE, NSTRIPE)])
        plsc.subcore_barrier()

        def start_v(j, b):
            pltpu.async_copy(vals_hbm.at[pl.ds((base + j) * CH, CH)],
                             vbuf.at[b], sv[b])

        def wait_v(b):
            pltpu.make_async_copy(vals_hbm.at[pl.ds(0, CH)], vbuf.at[b],
                                  sv[b]).wait()

        def start_s(j, b):
            pltpu.async_copy(vbuf.at[b], acc.at[idxall.at[j]], ss[b], add=True)

        def wait_s(b):
            pltpu.make_async_copy(vbuf.at[b], acc.at[idxall.at[0]],
                                  ss[b]).wait()

        for b in range(NB):
            start_v(b, b)

        def body(g, carry):
            for b in range(NB):
                j = g * NB + b
                wait_v(b)
                start_s(j, b)
                wait_s(b)
                start_v(j + NB, b)
            return carry

        lax.fori_loop(0, NGROUPS - 1, body, 0)
        for b in range(NB):
            j = (NGROUPS - 1) * NB + b
            wait_v(b)
            start_s(j, b)
            wait_s(b)

        plsc.subcore_barrier()
        pltpu.sync_copy(acc.at[pl.ds(s * NSTRIPE, NSTRIPE)], buf_v)
        pltpu.sync_copy(buf_v, out_hbm.at[pl.ds(c * N_ACC + s * NSTRIPE,
                                                NSTRIPE)])

    return sc_gather, sc_scatter


def _sc_gather(h, src2d):
    return _build_sc_kernels()[0](h, src2d)


def _sc_scatter(vals, dst2d, zeros_nh):
    return _build_sc_kernels()[1](vals, dst2d, zeros_nh)


def kernel(x_node, x_edge, edge_index, node_W, node_b, edge_W, edge_b,
           en_W1, en_b1, en_W2, en_b2, gru_Wih, gru_Whh, gru_bih, gru_bhh):
    pad = E_PAD - E
    src = jnp.concatenate([edge_index[0], jnp.zeros((pad,), jnp.int32)])
    dst = jnp.concatenate([edge_index[1], jnp.full((pad,), N, jnp.int32)])
    src = src.reshape(NCHUNK_P, CH)
    dst = dst.reshape(NCHUNK_P, CH)
    x_edge_p = jnp.concatenate(
        [x_edge, jnp.zeros((pad, x_edge.shape[1]), _f32)])

    node_WT = node_W.T
    node_b2d = node_b.reshape(1, H)
    edge_WT = edge_W.T
    edge_b2d = edge_b.reshape(1, H)
    en_W1T = en_W1.T
    en_b12d = en_b1.reshape(1, H)
    b2c = en_b2.reshape(HH, 1)

    wir = gru_Wih[0:H].T
    wiz = gru_Wih[H:2 * H].T
    win = gru_Wih[2 * H:3 * H].T
    whr = gru_Whh[0:H].T
    whz = gru_Whh[H:2 * H].T
    whn = gru_Whh[2 * H:3 * H].T
    br = (gru_bih[0:H] + gru_bhh[0:H]).reshape(1, H)
    bz = (gru_bih[H:2 * H] + gru_bhh[H:2 * H]).reshape(1, H)
    bin_ = gru_bih[2 * H:3 * H].reshape(1, H)
    bhn = gru_bhh[2 * H:3 * H].reshape(1, H)
    mats = (wir, wiz, win, whr, whz, whn)
    biases = (br, bz, bin_, bhn)

    zeros_nh = jnp.zeros((N_ACC, H), _f32)
    ones_eh = jnp.ones((E_PAD, H), _f32)

    h = _embed_node(x_node, node_WT, node_b2d)
    ef = _edge_feat(x_edge_p, edge_WT, edge_b2d, en_W1T, en_b12d)

    degp = _sc_scatter(ones_eh, dst, zeros_nh)
    d0, d1 = degp[:N], degp[N_ACC:N_ACC + N]

    for _ in range(N_LAYERS):
        hs = _sc_gather(h, src)
        msg = _msg(hs, ef, en_W2, b2c)
        mp = _sc_scatter(msg, dst, zeros_nh)
        h = _gru(mp[:N], mp[N_ACC:N_ACC + N], d0, d1, h, mats, biases)
    return h
